# baseline (speedup 1.0000x reference)
import functools

import jax
import jax.numpy as jnp
from jax import lax
from jax.experimental import pallas as pl
from jax.experimental.pallas import tpu as pltpu
from jax.experimental.pallas import tpu_sc as plsc

N = 10000
E = 320000
D_IN = 128
HID = 156
FN = 4
D_OUT = 2

NC = 2
NS = 16
NP = 10240
DP = 160
DPH = DP // 2
CH = 100
PER_TILE = E // NS
NCH = PER_TILE // CH
PER_TILE_DEG = E // (NC * NS)
NCH_DEG = PER_TILE_DEG // CH
RPT = NP // NS
WCH = 64
NWC = RPT // WCH
NB = 4

_mesh = plsc.VectorSubcoreMesh(
    core_axis_name="c", subcore_axis_name="s", num_cores=NC, num_subcores=NS)
_sc_params = pltpu.CompilerParams(use_tc_tiling_on_sc=False)


def _zero_fill(zbuf, rows, width):
    per_row = width // 16
    zv = jnp.zeros((16,), jnp.float32)

    def body(i, _):
        r = i // per_row
        cix = (i % per_row) * 16
        zbuf[r, pl.ds(cix, 16)] = zv
        return 0

    lax.fori_loop(0, rows * per_row, body, 0)


def _zero_shared(shared, zbuf, sidx):
    def zchunk(j, _):
        pltpu.sync_copy(zbuf, shared.at[pl.ds(sidx * RPT + j * WCH, WCH)])
        return 0

    lax.fori_loop(0, NWC, zchunk, 0)


def _drain_shared(shared, zbuf, out, sidx):
    def wchunk(j, _):
        start = sidx * RPT + j * WCH
        pltpu.sync_copy(shared.at[pl.ds(start, WCH)], zbuf)
        pltpu.sync_copy(zbuf, out.at[pl.ds(start, WCH)])
        return 0

    lax.fori_loop(0, NWC, wchunk, 0)


def _deg_body(rowi2, out0, out1, idx_c, ones_buf, zbuf, shared, dsem):
    cidx = lax.axis_index("c")
    sidx = lax.axis_index("s")
    wid = sidx * NC + cidx

    lane = lax.broadcasted_iota(jnp.int32, (16,), 0)
    onehot = jnp.where(lane == 0, 1.0, 0.0).astype(jnp.float32)

    def fill(i, _):
        ones_buf[i, pl.ds(0, 16)] = onehot
        return 0

    lax.fori_loop(0, CH, fill, 0)
    _zero_fill(zbuf, WCH, 16)
    _zero_shared(shared, zbuf, sidx)
    pltpu.sync_copy(rowi2.at[pl.ds(wid * NCH_DEG, NCH_DEG)], idx_c)
    plsc.subcore_barrier()

    def echunk(k, _):
        @pl.when(k >= 8)
        def _():
            pltpu.make_async_copy(
                ones_buf, shared.at[idx_c.at[0]], dsem).wait()

        pltpu.async_copy(ones_buf, shared.at[idx_c.at[k]], dsem, add=True)
        return 0

    lax.fori_loop(0, NCH_DEG, echunk, 0)

    def edrain(k, _):
        pltpu.make_async_copy(ones_buf, shared.at[idx_c.at[0]], dsem).wait()
        return 0

    lax.fori_loop(0, 8, edrain, 0)
    plsc.subcore_barrier()

    @pl.when(cidx == 0)
    def _():
        _drain_shared(shared, zbuf, out0, sidx)

    @pl.when(cidx == 1)
    def _():
        _drain_shared(shared, zbuf, out1, sidx)


_deg_kernel = functools.partial(
    pl.kernel,
    out_type=(
        jax.ShapeDtypeStruct((NP, 16), jnp.float32),
        jax.ShapeDtypeStruct((NP, 16), jnp.float32),
    ),
    mesh=_mesh,
    scratch_types=[
        pltpu.VMEM((NCH_DEG, CH), jnp.int32),
        pltpu.VMEM((CH, 16), jnp.float32),
        pltpu.VMEM((WCH, 16), jnp.float32),
        pltpu.VMEM_SHARED((NP, 16), jnp.float32),
        pltpu.SemaphoreType.DMA,
    ],
    compiler_params=_sc_params,
    name="deg_sc",
)(_deg_body)


def _edge_ring(table, shared, idx_r, idx_c, rbufs, gsems, ssems):
    rounds = NCH // NB

    for b in range(NB):
        pltpu.async_copy(table.at[idx_r.at[b]], rbufs[b], gsems[b])

    def round_body(g, _):
        base = g * NB
        for b in range(NB):
            pltpu.make_async_copy(
                table.at[idx_r.at[base + b]], rbufs[b], gsems[b]).wait()
            pltpu.async_copy(
                rbufs[b], shared.at[idx_c.at[base + b]], ssems[b], add=True)
        for b in range(NB):
            pltpu.make_async_copy(
                rbufs[b], shared.at[idx_c.at[base + b]], ssems[b]).wait()

            @pl.when(g < rounds - 1)
            def _(b=b):
                pltpu.async_copy(
                    table.at[idx_r.at[base + NB + b]], rbufs[b], gsems[b])

        return 0

    lax.fori_loop(0, rounds, round_body, 0)


def _scale_rows(zbuf, dinvbuf, j, square):
    def sgroup(g, _):
        d16 = dinvbuf[pl.ds(j * WCH + g * 16, 16)]
        if square:
            d16 = d16 * d16
        for i in range(16):
            d = d16[i]
            r = g * 16 + i
            for c in range(DPH // 16):
                sl = pl.ds(c * 16, 16)
                zbuf[r, sl] = zbuf[r, sl] * d
        return 0

    lax.fori_loop(0, WCH // 16, sgroup, 0)


def _spmm2x_half(table, dinv1d, v_out, w_out, u2_out, rowi2, coli2,
                 idx_r, idx_c, rbufs, zbuf, dinvbuf, shared,
                 gsems, ssems, sidx):
    _zero_fill(zbuf, WCH, DPH)
    _zero_shared(shared, zbuf, sidx)
    pltpu.sync_copy(rowi2.at[pl.ds(sidx * NCH, NCH)], idx_r)
    pltpu.sync_copy(coli2.at[pl.ds(sidx * NCH, NCH)], idx_c)
    pltpu.sync_copy(dinv1d.at[pl.ds(sidx * RPT, RPT)], dinvbuf)
    plsc.subcore_barrier()

    _edge_ring(table, shared, idx_r, idx_c, rbufs, gsems, ssems)
    plsc.subcore_barrier()

    def dchunk(j, _):
        start = sidx * RPT + j * WCH
        pltpu.sync_copy(shared.at[pl.ds(start, WCH)], zbuf)
        pltpu.sync_copy(zbuf, v_out.at[pl.ds(start, WCH)])
        _scale_rows(zbuf, dinvbuf, j, square=True)
        pltpu.sync_copy(zbuf, u2_out.at[pl.ds(start, WCH)])
        return 0

    lax.fori_loop(0, NWC, dchunk, 0)

    _zero_fill(zbuf, WCH, DPH)
    _zero_shared(shared, zbuf, sidx)
    plsc.subcore_barrier()

    _edge_ring(u2_out, shared, idx_r, idx_c, rbufs, gsems, ssems)
    plsc.subcore_barrier()
    _drain_shared(shared, zbuf, w_out, sidx)


@functools.partial(
    pl.kernel,
    out_type=(
        jax.ShapeDtypeStruct((NP, DPH), jnp.float32),
        jax.ShapeDtypeStruct((NP, DPH), jnp.float32),
        jax.ShapeDtypeStruct((NP, DPH), jnp.float32),
        jax.ShapeDtypeStruct((NP, DPH), jnp.float32),
        jax.ShapeDtypeStruct((NP, DPH), jnp.float32),
        jax.ShapeDtypeStruct((NP, DPH), jnp.float32),
    ),
    mesh=_mesh,
    scratch_types=[
        pltpu.VMEM((NCH, CH), jnp.int32),
        pltpu.VMEM((NCH, CH), jnp.int32),
        [pltpu.VMEM((CH, DPH), jnp.float32) for _ in range(NB)],
        pltpu.VMEM((WCH, DPH), jnp.float32),
        pltpu.VMEM((RPT,), jnp.float32),
        pltpu.VMEM_SHARED((NP, DPH), jnp.float32),
        [pltpu.SemaphoreType.DMA for _ in range(NB)],
        [pltpu.SemaphoreType.DMA for _ in range(NB)],
    ],
    compiler_params=_sc_params,
    name="spmm2x_sc",
)
def _spmm2x_kernel(t_lo, t_hi, rowi2, coli2, dinv1d,
                   v_lo, v_hi, w_lo, w_hi, u2_lo, u2_hi,
                   idx_r, idx_c, rbufs, zbuf, dinvbuf, shared, gsems, ssems):
    cidx = lax.axis_index("c")
    sidx = lax.axis_index("s")

    @pl.when(cidx == 0)
    def _():
        _spmm2x_half(t_lo, dinv1d, v_lo, w_lo, u2_lo, rowi2, coli2,
                     idx_r, idx_c, rbufs, zbuf, dinvbuf, shared,
                     gsems, ssems, sidx)

    @pl.when(cidx == 1)
    def _():
        _spmm2x_half(t_hi, dinv1d, v_hi, w_hi, u2_hi, rowi2, coli2,
                     idx_r, idx_c, rbufs, zbuf, dinvbuf, shared,
                     gsems, ssems, sidx)


BN = 1000
GRID = N // BN


def _h0_body(x_ref, w1_ref, b1_ref, w2_ref, b2_ref, h0_ref):
    h = jnp.maximum(
        jnp.dot(x_ref[...], w1_ref[...], preferred_element_type=jnp.float32)
        + b1_ref[...], 0.0)
    h0_ref[...] = (
        jnp.dot(h, w2_ref[...], preferred_element_type=jnp.float32)
        + b2_ref[...])


_h0_kernel = pl.pallas_call(
    _h0_body,
    grid=(GRID,),
    in_specs=[
        pl.BlockSpec((BN, D_IN), lambda i: (i, 0)),
        pl.BlockSpec((D_IN, HID), lambda i: (0, 0)),
        pl.BlockSpec((1, HID), lambda i: (0, 0)),
        pl.BlockSpec((HID, HID), lambda i: (0, 0)),
        pl.BlockSpec((1, HID), lambda i: (0, 0)),
    ],
    out_specs=pl.BlockSpec((BN, HID), lambda i: (i, 0)),
    out_shape=jax.ShapeDtypeStruct((N, HID), jnp.float32),
)


def _u_body(h0_ref, d0_ref, d1_ref, ulo_ref, uhi_ref, dinv_ref):
    h0 = h0_ref[...]
    deg = d0_ref[:, :1] + d1_ref[:, :1]
    dinv = jnp.where(deg > 0, lax.rsqrt(jnp.where(deg > 0, deg, 1.0)), 0.0)
    u = dinv * h0
    dinv_ref[...] = dinv
    ulo_ref[...] = u[:, :DPH]
    uhi_ref[...] = jnp.concatenate(
        [u[:, DPH:], jnp.zeros((u.shape[0], DP - HID), jnp.float32)], axis=1)


_u_kernel = pl.pallas_call(
    _u_body,
    grid=(GRID,),
    in_specs=[
        pl.BlockSpec((BN, HID), lambda i: (i, 0)),
        pl.BlockSpec((BN, 16), lambda i: (i, 0)),
        pl.BlockSpec((BN, 16), lambda i: (i, 0)),
    ],
    out_specs=[
        pl.BlockSpec((BN, DPH), lambda i: (i, 0)),
        pl.BlockSpec((BN, DPH), lambda i: (i, 0)),
        pl.BlockSpec((BN, 1), lambda i: (i, 0)),
    ],
    out_shape=[
        jax.ShapeDtypeStruct((N, DPH), jnp.float32),
        jax.ShapeDtypeStruct((N, DPH), jnp.float32),
        jax.ShapeDtypeStruct((N, 1), jnp.float32),
    ],
)


def _head_body(h0_ref, vlo_ref, vhi_ref, wlo_ref, whi_ref, dinv_ref,
               th_ref, fb_ref, wf_ref, bf_ref, wx_ref, bx_ref,
               wc_ref, bc_ref, y_ref):
    h0 = h0_ref[...]
    dinv = dinv_ref[...]
    wf = wf_ref[...]
    p0 = dinv * jnp.concatenate(
        [vlo_ref[...], vhi_ref[...]], axis=1)[:, :HID]
    q = dinv * jnp.concatenate(
        [wlo_ref[...], whi_ref[...]], axis=1)[:, :HID]

    xp = jnp.tanh(
        jnp.dot(h0, wx_ref[...], preferred_element_type=jnp.float32)
        + bx_ref[...])

    b0 = h0 + 2.0 * p0 + q
    b1 = h0 - q
    b2 = h0 - 2.0 * p0 + q

    cks = (0.25, 0.5, 0.25)
    hs = []
    ls = []
    for f in range(FN):
        t0 = jnp.maximum(th_ref[f, 0], 0.0) * cks[0]
        t1 = jnp.maximum(th_ref[f, 1], 0.0) * cks[1]
        t2 = jnp.maximum(th_ref[f, 2], 0.0) * cks[2]
        hf = t0 * b0 + t1 * b1 + t2 * b2 + fb_ref[f:f + 1, :]
        hp = jnp.tanh(
            jnp.dot(hf, wf, preferred_element_type=jnp.float32)
            + bf_ref[...])
        ls.append(jnp.sum(hp * xp, axis=1, keepdims=True))
        hs.append(hf)
    m = jnp.maximum(jnp.maximum(ls[0], ls[1]), jnp.maximum(ls[2], ls[3]))
    es = [jnp.exp(l - m) for l in ls]
    tot = es[0] + es[1] + es[2] + es[3]
    res = (es[0] * hs[0] + es[1] * hs[1] + es[2] * hs[2] + es[3] * hs[3]) / tot
    y_ref[...] = (jnp.dot(res, wc_ref[...], preferred_element_type=jnp.float32)
                  + bc_ref[...])


_head_kernel = pl.pallas_call(
    _head_body,
    grid=(GRID,),
    in_specs=[
        pl.BlockSpec((BN, HID), lambda i: (i, 0)),
        pl.BlockSpec((BN, DPH), lambda i: (i, 0)),
        pl.BlockSpec((BN, DPH), lambda i: (i, 0)),
        pl.BlockSpec((BN, DPH), lambda i: (i, 0)),
        pl.BlockSpec((BN, DPH), lambda i: (i, 0)),
        pl.BlockSpec((BN, 1), lambda i: (i, 0)),
        pl.BlockSpec(memory_space=pltpu.SMEM),
        pl.BlockSpec((FN, HID), lambda i: (0, 0)),
        pl.BlockSpec((HID, HID), lambda i: (0, 0)),
        pl.BlockSpec((1, HID), lambda i: (0, 0)),
        pl.BlockSpec((HID, HID), lambda i: (0, 0)),
        pl.BlockSpec((1, HID), lambda i: (0, 0)),
        pl.BlockSpec((HID, D_OUT), lambda i: (0, 0)),
        pl.BlockSpec((1, D_OUT), lambda i: (0, 0)),
    ],
    out_specs=pl.BlockSpec((BN, D_OUT), lambda i: (i, 0)),
    out_shape=jax.ShapeDtypeStruct((N, D_OUT), jnp.float32),
)


def kernel(x, edge_index, W1, b1, W2, b2, thetas, fbias, Wf, bf, Wx, bx, Wc, bc):
    row2 = edge_index[0].reshape(E // CH, CH)
    col2 = edge_index[1].reshape(E // CH, CH)

    deg0, deg1 = _deg_kernel(row2)

    h0 = _h0_kernel(x, W1, b1.reshape(1, HID), W2, b2.reshape(1, HID))
    u_lo, u_hi, dinv = _u_kernel(h0, deg0, deg1)

    dinv_pad = jnp.pad(dinv[:, 0], (0, NP - N))
    v_lo, v_hi, w_lo, w_hi, _, _ = _spmm2x_kernel(
        u_lo, u_hi, row2, col2, dinv_pad)

    y = _head_kernel(
        h0, v_lo, v_hi, w_lo, w_hi, dinv, thetas, fbias,
        Wf, bf.reshape(1, HID), Wx, bx.reshape(1, HID),
        Wc, bc.reshape(1, D_OUT))
    return y


# speedup vs baseline: 1.0258x; 1.0258x over previous
import functools

import jax
import jax.numpy as jnp
from jax import lax
from jax.experimental import pallas as pl
from jax.experimental.pallas import tpu as pltpu
from jax.experimental.pallas import tpu_sc as plsc

N = 10000
E = 320000
D_IN = 128
HID = 156
FN = 4
D_OUT = 2

NC = 2
NS = 16
NP = 10240
DP = 160
DPH = DP // 2
CH = 80
PER_TILE = E // NS
NCH = PER_TILE // CH
PER_TILE_DEG = E // (NC * NS)
NCH_DEG = PER_TILE_DEG // CH
RPT = NP // NS
WCH = 64
NWC = RPT // WCH
NB = 5

_mesh = plsc.VectorSubcoreMesh(
    core_axis_name="c", subcore_axis_name="s", num_cores=NC, num_subcores=NS)
_sc_params = pltpu.CompilerParams(use_tc_tiling_on_sc=False)


def _zero_fill(zbuf, rows, width):
    per_row = width // 16
    zv = jnp.zeros((16,), jnp.float32)

    def body(i, _):
        r = i // per_row
        cix = (i % per_row) * 16
        zbuf[r, pl.ds(cix, 16)] = zv
        return 0

    lax.fori_loop(0, rows * per_row, body, 0)


def _zero_shared(shared, zbuf, sidx):
    def zchunk(j, _):
        pltpu.sync_copy(zbuf, shared.at[pl.ds(sidx * RPT + j * WCH, WCH)])
        return 0

    lax.fori_loop(0, NWC, zchunk, 0)


def _drain_shared(shared, zbuf, out, sidx):
    def wchunk(j, _):
        start = sidx * RPT + j * WCH
        pltpu.sync_copy(shared.at[pl.ds(start, WCH)], zbuf)
        pltpu.sync_copy(zbuf, out.at[pl.ds(start, WCH)])
        return 0

    lax.fori_loop(0, NWC, wchunk, 0)


def _deg_body(rowi2, out0, out1, idx_c, ones_buf, zbuf, shared, dsem):
    cidx = lax.axis_index("c")
    sidx = lax.axis_index("s")
    wid = sidx * NC + cidx

    lane = lax.broadcasted_iota(jnp.int32, (16,), 0)
    onehot = jnp.where(lane == 0, 1.0, 0.0).astype(jnp.float32)

    def fill(i, _):
        ones_buf[i, pl.ds(0, 16)] = onehot
        return 0

    lax.fori_loop(0, CH, fill, 0)
    _zero_fill(zbuf, WCH, 16)
    _zero_shared(shared, zbuf, sidx)
    pltpu.sync_copy(rowi2.at[pl.ds(wid * NCH_DEG, NCH_DEG)], idx_c)
    plsc.subcore_barrier()

    def echunk(k, _):
        @pl.when(k >= 8)
        def _():
            pltpu.make_async_copy(
                ones_buf, shared.at[idx_c.at[0]], dsem).wait()

        pltpu.async_copy(ones_buf, shared.at[idx_c.at[k]], dsem, add=True)
        return 0

    lax.fori_loop(0, NCH_DEG, echunk, 0)

    def edrain(k, _):
        pltpu.make_async_copy(ones_buf, shared.at[idx_c.at[0]], dsem).wait()
        return 0

    lax.fori_loop(0, 8, edrain, 0)
    plsc.subcore_barrier()

    @pl.when(cidx == 0)
    def _():
        _drain_shared(shared, zbuf, out0, sidx)

    @pl.when(cidx == 1)
    def _():
        _drain_shared(shared, zbuf, out1, sidx)


_deg_kernel = functools.partial(
    pl.kernel,
    out_type=(
        jax.ShapeDtypeStruct((NP, 16), jnp.float32),
        jax.ShapeDtypeStruct((NP, 16), jnp.float32),
    ),
    mesh=_mesh,
    scratch_types=[
        pltpu.VMEM((NCH_DEG, CH), jnp.int32),
        pltpu.VMEM((CH, 16), jnp.float32),
        pltpu.VMEM((WCH, 16), jnp.float32),
        pltpu.VMEM_SHARED((NP, 16), jnp.float32),
        pltpu.SemaphoreType.DMA,
    ],
    compiler_params=_sc_params,
    name="deg_sc",
)(_deg_body)


def _edge_ring(table, shared, idx_r, idx_c, rbufs, gsems, ssems):
    rounds = NCH // NB

    for b in range(NB):
        pltpu.async_copy(table.at[idx_r.at[b]], rbufs[b], gsems[b])

    def round_body(g, _):
        base = g * NB
        for b in range(NB):
            pltpu.make_async_copy(
                table.at[idx_r.at[base + b]], rbufs[b], gsems[b]).wait()
            pltpu.async_copy(
                rbufs[b], shared.at[idx_c.at[base + b]], ssems[b], add=True)
        for b in range(NB):
            pltpu.make_async_copy(
                rbufs[b], shared.at[idx_c.at[base + b]], ssems[b]).wait()

            @pl.when(g < rounds - 1)
            def _(b=b):
                pltpu.async_copy(
                    table.at[idx_r.at[base + NB + b]], rbufs[b], gsems[b])

        return 0

    lax.fori_loop(0, rounds, round_body, 0)


def _scale_rows(zbuf, dinvbuf, j, square):
    def sgroup(g, _):
        d16 = dinvbuf[pl.ds(j * WCH + g * 16, 16)]
        if square:
            d16 = d16 * d16
        for i in range(16):
            d = d16[i]
            r = g * 16 + i
            for c in range(DPH // 16):
                sl = pl.ds(c * 16, 16)
                zbuf[r, sl] = zbuf[r, sl] * d
        return 0

    lax.fori_loop(0, WCH // 16, sgroup, 0)


def _spmm2x_half(table, dinv1d, v_out, w_out, u2_out, rowi2, coli2,
                 idx_r, idx_c, rbufs, zbuf, dinvbuf, shared,
                 gsems, ssems, sidx):
    _zero_fill(zbuf, WCH, DPH)
    _zero_shared(shared, zbuf, sidx)
    pltpu.sync_copy(rowi2.at[pl.ds(sidx * NCH, NCH)], idx_r)
    pltpu.sync_copy(coli2.at[pl.ds(sidx * NCH, NCH)], idx_c)
    pltpu.sync_copy(dinv1d.at[pl.ds(sidx * RPT, RPT)], dinvbuf)
    plsc.subcore_barrier()

    _edge_ring(table, shared, idx_r, idx_c, rbufs, gsems, ssems)
    plsc.subcore_barrier()

    def dchunk(j, _):
        start = sidx * RPT + j * WCH
        pltpu.sync_copy(shared.at[pl.ds(start, WCH)], zbuf)
        pltpu.sync_copy(zbuf, v_out.at[pl.ds(start, WCH)])
        _scale_rows(zbuf, dinvbuf, j, square=True)
        pltpu.sync_copy(zbuf, u2_out.at[pl.ds(start, WCH)])
        return 0

    lax.fori_loop(0, NWC, dchunk, 0)

    _zero_fill(zbuf, WCH, DPH)
    _zero_shared(shared, zbuf, sidx)
    plsc.subcore_barrier()

    _edge_ring(u2_out, shared, idx_r, idx_c, rbufs, gsems, ssems)
    plsc.subcore_barrier()
    _drain_shared(shared, zbuf, w_out, sidx)


@functools.partial(
    pl.kernel,
    out_type=(
        jax.ShapeDtypeStruct((NP, DPH), jnp.float32),
        jax.ShapeDtypeStruct((NP, DPH), jnp.float32),
        jax.ShapeDtypeStruct((NP, DPH), jnp.float32),
        jax.ShapeDtypeStruct((NP, DPH), jnp.float32),
        jax.ShapeDtypeStruct((NP, DPH), jnp.float32),
        jax.ShapeDtypeStruct((NP, DPH), jnp.float32),
    ),
    mesh=_mesh,
    scratch_types=[
        pltpu.VMEM((NCH, CH), jnp.int32),
        pltpu.VMEM((NCH, CH), jnp.int32),
        [pltpu.VMEM((CH, DPH), jnp.float32) for _ in range(NB)],
        pltpu.VMEM((WCH, DPH), jnp.float32),
        pltpu.VMEM((RPT,), jnp.float32),
        pltpu.VMEM_SHARED((NP, DPH), jnp.float32),
        [pltpu.SemaphoreType.DMA for _ in range(NB)],
        [pltpu.SemaphoreType.DMA for _ in range(NB)],
    ],
    compiler_params=_sc_params,
    name="spmm2x_sc",
)
def _spmm2x_kernel(t_lo, t_hi, rowi2, coli2, dinv1d,
                   v_lo, v_hi, w_lo, w_hi, u2_lo, u2_hi,
                   idx_r, idx_c, rbufs, zbuf, dinvbuf, shared, gsems, ssems):
    cidx = lax.axis_index("c")
    sidx = lax.axis_index("s")

    @pl.when(cidx == 0)
    def _():
        _spmm2x_half(t_lo, dinv1d, v_lo, w_lo, u2_lo, rowi2, coli2,
                     idx_r, idx_c, rbufs, zbuf, dinvbuf, shared,
                     gsems, ssems, sidx)

    @pl.when(cidx == 1)
    def _():
        _spmm2x_half(t_hi, dinv1d, v_hi, w_hi, u2_hi, rowi2, coli2,
                     idx_r, idx_c, rbufs, zbuf, dinvbuf, shared,
                     gsems, ssems, sidx)


BN = 1000
GRID = N // BN


def _h0_body(x_ref, w1_ref, b1_ref, w2_ref, b2_ref, h0_ref):
    h = jnp.maximum(
        jnp.dot(x_ref[...], w1_ref[...], preferred_element_type=jnp.float32)
        + b1_ref[...], 0.0)
    h0_ref[...] = (
        jnp.dot(h, w2_ref[...], preferred_element_type=jnp.float32)
        + b2_ref[...])


_h0_kernel = pl.pallas_call(
    _h0_body,
    grid=(GRID,),
    in_specs=[
        pl.BlockSpec((BN, D_IN), lambda i: (i, 0)),
        pl.BlockSpec((D_IN, HID), lambda i: (0, 0)),
        pl.BlockSpec((1, HID), lambda i: (0, 0)),
        pl.BlockSpec((HID, HID), lambda i: (0, 0)),
        pl.BlockSpec((1, HID), lambda i: (0, 0)),
    ],
    out_specs=pl.BlockSpec((BN, HID), lambda i: (i, 0)),
    out_shape=jax.ShapeDtypeStruct((N, HID), jnp.float32),
)


def _u_body(h0_ref, d0_ref, d1_ref, ulo_ref, uhi_ref, dinv_ref):
    h0 = h0_ref[...]
    deg = d0_ref[:, :1] + d1_ref[:, :1]
    dinv = jnp.where(deg > 0, lax.rsqrt(jnp.where(deg > 0, deg, 1.0)), 0.0)
    u = dinv * h0
    dinv_ref[...] = dinv
    ulo_ref[...] = u[:, :DPH]
    uhi_ref[...] = jnp.concatenate(
        [u[:, DPH:], jnp.zeros((u.shape[0], DP - HID), jnp.float32)], axis=1)


_u_kernel = pl.pallas_call(
    _u_body,
    grid=(GRID,),
    in_specs=[
        pl.BlockSpec((BN, HID), lambda i: (i, 0)),
        pl.BlockSpec((BN, 16), lambda i: (i, 0)),
        pl.BlockSpec((BN, 16), lambda i: (i, 0)),
    ],
    out_specs=[
        pl.BlockSpec((BN, DPH), lambda i: (i, 0)),
        pl.BlockSpec((BN, DPH), lambda i: (i, 0)),
        pl.BlockSpec((BN, 1), lambda i: (i, 0)),
    ],
    out_shape=[
        jax.ShapeDtypeStruct((N, DPH), jnp.float32),
        jax.ShapeDtypeStruct((N, DPH), jnp.float32),
        jax.ShapeDtypeStruct((N, 1), jnp.float32),
    ],
)


def _head_body(h0_ref, vlo_ref, vhi_ref, wlo_ref, whi_ref, dinv_ref,
               th_ref, fb_ref, wf_ref, bf_ref, wx_ref, bx_ref,
               wc_ref, bc_ref, y_ref):
    h0 = h0_ref[...]
    dinv = dinv_ref[...]
    wf = wf_ref[...]
    p0 = dinv * jnp.concatenate(
        [vlo_ref[...], vhi_ref[...]], axis=1)[:, :HID]
    q = dinv * jnp.concatenate(
        [wlo_ref[...], whi_ref[...]], axis=1)[:, :HID]

    xp = jnp.tanh(
        jnp.dot(h0, wx_ref[...], preferred_element_type=jnp.float32)
        + bx_ref[...])

    b0 = h0 + 2.0 * p0 + q
    b1 = h0 - q
    b2 = h0 - 2.0 * p0 + q

    cks = (0.25, 0.5, 0.25)
    hs = []
    ls = []
    for f in range(FN):
        t0 = jnp.maximum(th_ref[f, 0], 0.0) * cks[0]
        t1 = jnp.maximum(th_ref[f, 1], 0.0) * cks[1]
        t2 = jnp.maximum(th_ref[f, 2], 0.0) * cks[2]
        hf = t0 * b0 + t1 * b1 + t2 * b2 + fb_ref[f:f + 1, :]
        hp = jnp.tanh(
            jnp.dot(hf, wf, preferred_element_type=jnp.float32)
            + bf_ref[...])
        ls.append(jnp.sum(hp * xp, axis=1, keepdims=True))
        hs.append(hf)
    m = jnp.maximum(jnp.maximum(ls[0], ls[1]), jnp.maximum(ls[2], ls[3]))
    es = [jnp.exp(l - m) for l in ls]
    tot = es[0] + es[1] + es[2] + es[3]
    res = (es[0] * hs[0] + es[1] * hs[1] + es[2] * hs[2] + es[3] * hs[3]) / tot
    y_ref[...] = (jnp.dot(res, wc_ref[...], preferred_element_type=jnp.float32)
                  + bc_ref[...])


_head_kernel = pl.pallas_call(
    _head_body,
    grid=(GRID,),
    in_specs=[
        pl.BlockSpec((BN, HID), lambda i: (i, 0)),
        pl.BlockSpec((BN, DPH), lambda i: (i, 0)),
        pl.BlockSpec((BN, DPH), lambda i: (i, 0)),
        pl.BlockSpec((BN, DPH), lambda i: (i, 0)),
        pl.BlockSpec((BN, DPH), lambda i: (i, 0)),
        pl.BlockSpec((BN, 1), lambda i: (i, 0)),
        pl.BlockSpec(memory_space=pltpu.SMEM),
        pl.BlockSpec((FN, HID), lambda i: (0, 0)),
        pl.BlockSpec((HID, HID), lambda i: (0, 0)),
        pl.BlockSpec((1, HID), lambda i: (0, 0)),
        pl.BlockSpec((HID, HID), lambda i: (0, 0)),
        pl.BlockSpec((1, HID), lambda i: (0, 0)),
        pl.BlockSpec((HID, D_OUT), lambda i: (0, 0)),
        pl.BlockSpec((1, D_OUT), lambda i: (0, 0)),
    ],
    out_specs=pl.BlockSpec((BN, D_OUT), lambda i: (i, 0)),
    out_shape=jax.ShapeDtypeStruct((N, D_OUT), jnp.float32),
)


def kernel(x, edge_index, W1, b1, W2, b2, thetas, fbias, Wf, bf, Wx, bx, Wc, bc):
    row2 = edge_index[0].reshape(E // CH, CH)
    col2 = edge_index[1].reshape(E // CH, CH)

    deg0, deg1 = _deg_kernel(row2)

    h0 = _h0_kernel(x, W1, b1.reshape(1, HID), W2, b2.reshape(1, HID))
    u_lo, u_hi, dinv = _u_kernel(h0, deg0, deg1)

    dinv_pad = jnp.pad(dinv[:, 0], (0, NP - N))
    v_lo, v_hi, w_lo, w_hi, _, _ = _spmm2x_kernel(
        u_lo, u_hi, row2, col2, dinv_pad)

    y = _head_kernel(
        h0, v_lo, v_hi, w_lo, w_hi, dinv, thetas, fbias,
        Wf, bf.reshape(1, HID), Wx, bx.reshape(1, HID),
        Wc, bc.reshape(1, D_OUT))
    return y
